# Initial kernel scaffold; baseline (speedup 1.0000x reference)
import jax
import jax.numpy as jnp
from jax.experimental import pallas as pl


def kernel(x, edge_index, batch, W1, b1, W2, b2, Wc1, bc1, Wc2, bc2):
    raise NotImplementedError("write your pallas kernel here")


# speedup vs baseline: 17.5900x; 17.5900x over previous
import functools

import jax
import jax.numpy as jnp
from jax import lax
from jax.experimental import pallas as pl
from jax.experimental.pallas import tpu as pltpu
from jax.experimental.pallas import tpu_sc as plsc

_N = 10000
_E = 320000
_D = 128
_H = 128
_HD = _H // 2
_FH = 256
_C = 5
_G = 64

_NSC = 2
_NTILE = 16
_NW = _NSC * _NTILE

_CH = 125
_NP = 10240
_RPT = _NP // _NTILE
_ZR = 128

_EPW_D = _E // _NW
_NCHUNK_D = _EPW_D // _CH
_EPT = _E // _NTILE
_NCHUNK = _EPT // _CH

_R = 2000
_NBLK = _N // _R

_sc_mesh = plsc.VectorSubcoreMesh(core_axis_name="c", subcore_axis_name="s")


@functools.partial(
    pl.kernel,
    out_type=jax.ShapeDtypeStruct((_NSC, _NP, 16), jnp.float32),
    mesh=_sc_mesh,
    compiler_params=pltpu.CompilerParams(use_tc_tiling_on_sc=False),
    scratch_types=[
        pltpu.VMEM((_NCHUNK_D, _CH), jnp.int32),
        pltpu.VMEM((_CH, 16), jnp.float32),
        pltpu.VMEM((_ZR, 16), jnp.float32),
        pltpu.VMEM_SHARED((_NP, 16), jnp.float32),
    ],
)
def _deg(dst_hbm, out_hbm, dstb, onesb, zb, acc):
    c = lax.axis_index("c")
    s = lax.axis_index("s")
    wid = c * _NTILE + s

    pltpu.sync_copy(dst_hbm.at[wid], dstb)

    def _fill(i, carry):
        onesb[i, :] = jnp.full((16,), 1.0, jnp.float32)
        zb[i, :] = jnp.zeros((16,), jnp.float32)
        return carry

    lax.fori_loop(0, _CH, _fill, 0)

    def _zero(k, carry):
        pltpu.sync_copy(zb, acc.at[pl.ds(s * _RPT + k * _ZR, _ZR)])
        return carry

    lax.fori_loop(0, _RPT // _ZR, _zero, 0)
    plsc.subcore_barrier()

    def _step(g, carry):
        pltpu.sync_copy(onesb, acc.at[dstb.at[g]], add=True)
        return carry

    lax.fori_loop(0, _NCHUNK_D, _step, 0)
    plsc.subcore_barrier()
    pltpu.sync_copy(acc.at[pl.ds(s * _RPT, _RPT)],
                    out_hbm.at[c, pl.ds(s * _RPT, _RPT)])


@functools.partial(
    pl.kernel,
    out_type=jax.ShapeDtypeStruct((_NSC, _NP, _HD), jnp.float32),
    mesh=_sc_mesh,
    compiler_params=pltpu.CompilerParams(use_tc_tiling_on_sc=False),
    scratch_types=[
        pltpu.VMEM((_NCHUNK, _CH), jnp.int32),
        pltpu.VMEM((_NCHUNK, _CH), jnp.int32),
        pltpu.VMEM((_CH, _HD), jnp.float32),
        pltpu.VMEM((_ZR, _HD), jnp.float32),
        pltpu.VMEM_SHARED((_NP, _HD), jnp.float32),
        pltpu.SemaphoreType.DMA,
    ],
)
def _mp(hp_hbm, src_hbm, dst_hbm, out_hbm, srcb, dstb, rows, zb, acc, sem):
    c = lax.axis_index("c")
    s = lax.axis_index("s")

    pltpu.sync_copy(src_hbm.at[s], srcb)
    pltpu.sync_copy(dst_hbm.at[s], dstb)

    def _fill(i, carry):
        for j in range(_HD // 16):
            zb[i, pl.ds(j * 16, 16)] = jnp.zeros((16,), jnp.float32)
        return carry

    lax.fori_loop(0, _ZR, _fill, 0)

    def _zero(k, carry):
        pltpu.sync_copy(zb, acc.at[pl.ds(s * _RPT + k * _ZR, _ZR)])
        return carry

    lax.fori_loop(0, _RPT // _ZR, _zero, 0)
    plsc.subcore_barrier()

    def _step(g, carry):
        pltpu.async_copy(hp_hbm.at[c].at[srcb.at[g]], rows, sem).wait()
        pltpu.sync_copy(rows, acc.at[dstb.at[g]], add=True)
        return carry

    lax.fori_loop(0, _NCHUNK, _step, 0)
    plsc.subcore_barrier()
    pltpu.sync_copy(acc.at[pl.ds(s * _RPT, _RPT)],
                    out_hbm.at[c, pl.ds(s * _RPT, _RPT)])


def _prep_body(x_ref, w_ref, degp_ref, hp_ref, dinv_ref):
    deg = degp_ref[0, :, 0:1] + degp_ref[1, :, 0:1] + 1.0
    dinv = lax.rsqrt(deg)
    h = jnp.dot(x_ref[...], w_ref[...], preferred_element_type=jnp.float32)
    hp = h * dinv
    hp_ref[0] = hp[:, 0:_HD]
    hp_ref[1] = hp[:, _HD:_H]
    dinv_ref[...] = dinv


def _prep_call(x, W1, degp):
    return pl.pallas_call(
        _prep_body,
        grid=(_NBLK,),
        in_specs=[
            pl.BlockSpec((_R, _D), lambda i: (i, 0)),
            pl.BlockSpec((_D, _H), lambda i: (0, 0)),
            pl.BlockSpec((_NSC, _R, 16), lambda i: (0, i, 0)),
        ],
        out_specs=[
            pl.BlockSpec((_NSC, _R, _HD), lambda i: (0, i, 0)),
            pl.BlockSpec((_R, 1), lambda i: (i, 0)),
        ],
        out_shape=[
            jax.ShapeDtypeStruct((_NSC, _N, _HD), jnp.float32),
            jax.ShapeDtypeStruct((_N, 1), jnp.float32),
        ],
    )(x, W1, degp)


def _mid_body(acc_ref, hp_ref, dinv_ref, b_ref, w_ref, out_ref):
    t = jnp.concatenate([acc_ref[0] + hp_ref[0], acc_ref[1] + hp_ref[1]],
                        axis=1)
    o = jnp.maximum(t * dinv_ref[...] + b_ref[...], 0.0)
    h2 = jnp.dot(o, w_ref[...], preferred_element_type=jnp.float32)
    hp2 = h2 * dinv_ref[...]
    out_ref[0] = hp2[:, 0:_HD]
    out_ref[1] = hp2[:, _HD:_H]


def _mid_call(acc, hp1, dinv, b1, W2):
    return pl.pallas_call(
        _mid_body,
        grid=(_NBLK,),
        in_specs=[
            pl.BlockSpec((_NSC, _R, _HD), lambda i: (0, i, 0)),
            pl.BlockSpec((_NSC, _R, _HD), lambda i: (0, i, 0)),
            pl.BlockSpec((_R, 1), lambda i: (i, 0)),
            pl.BlockSpec((1, _H), lambda i: (0, 0)),
            pl.BlockSpec((_H, _H), lambda i: (0, 0)),
        ],
        out_specs=pl.BlockSpec((_NSC, _R, _HD), lambda i: (0, i, 0)),
        out_shape=jax.ShapeDtypeStruct((_NSC, _N, _HD), jnp.float32),
    )(acc, hp1, dinv, b1, W2)


def _final_body(acc_ref, hp_ref, dinv_ref, b_ref, batch_ref, wc1_ref,
                bc1_ref, wc2_ref, bc2_ref, out_ref, pooled, counts):
    i = pl.program_id(0)

    @pl.when(i == 0)
    def _():
        pooled[...] = jnp.zeros_like(pooled)
        counts[...] = jnp.zeros_like(counts)

    t = jnp.concatenate([acc_ref[0] + hp_ref[0], acc_ref[1] + hp_ref[1]],
                        axis=1)
    o = jnp.maximum(t * dinv_ref[...] + b_ref[...], 0.0)
    gids = lax.broadcasted_iota(jnp.int32, (_R, _G), 1)
    mask = (batch_ref[...] == gids).astype(jnp.float32)
    dn = (((0,), (0,)), ((), ()))
    pooled[...] += lax.dot_general(mask, o, dn,
                                   preferred_element_type=jnp.float32)
    counts[...] += lax.dot_general(mask, jnp.ones((_R, 1), jnp.float32), dn,
                                   preferred_element_type=jnp.float32)

    @pl.when(i == pl.num_programs(0) - 1)
    def _():
        pm = pooled[...] / jnp.maximum(counts[...], 1.0)
        z = jnp.maximum(
            jnp.dot(pm, wc1_ref[...], preferred_element_type=jnp.float32)
            + bc1_ref[...], 0.0)
        out_ref[...] = (jnp.dot(z, wc2_ref[...],
                                preferred_element_type=jnp.float32)
                        + bc2_ref[...])


def _final_call(acc, hp2, dinv, b2, batch2, Wc1, bc1, Wc2, bc2):
    return pl.pallas_call(
        _final_body,
        grid=(_NBLK,),
        in_specs=[
            pl.BlockSpec((_NSC, _R, _HD), lambda i: (0, i, 0)),
            pl.BlockSpec((_NSC, _R, _HD), lambda i: (0, i, 0)),
            pl.BlockSpec((_R, 1), lambda i: (i, 0)),
            pl.BlockSpec((1, _H), lambda i: (0, 0)),
            pl.BlockSpec((_R, 1), lambda i: (i, 0)),
            pl.BlockSpec((_H, _FH), lambda i: (0, 0)),
            pl.BlockSpec((1, _FH), lambda i: (0, 0)),
            pl.BlockSpec((_FH, _C), lambda i: (0, 0)),
            pl.BlockSpec((1, _C), lambda i: (0, 0)),
        ],
        out_specs=pl.BlockSpec((_G, _C), lambda i: (0, 0)),
        out_shape=jax.ShapeDtypeStruct((_G, _C), jnp.float32),
        scratch_shapes=[
            pltpu.VMEM((_G, _H), jnp.float32),
            pltpu.VMEM((_G, 1), jnp.float32),
        ],
    )(acc, hp2, dinv, b2, batch2, Wc1, bc1, Wc2, bc2)


def kernel(x, edge_index, batch, W1, b1, W2, b2, Wc1, bc1, Wc2, bc2):
    src_t = edge_index[0].reshape(_NTILE, _NCHUNK, _CH)
    dst_t = edge_index[1].reshape(_NTILE, _NCHUNK, _CH)
    dst_w = edge_index[1].reshape(_NW, _NCHUNK_D, _CH)

    degp = _deg(dst_w)
    hp1, dinv = _prep_call(x, W1, degp)
    acc1 = _mp(hp1, src_t, dst_t)
    hp2 = _mid_call(acc1, hp1, dinv, b1.reshape(1, _H), W2)
    acc2 = _mp(hp2, src_t, dst_t)
    return _final_call(acc2, hp2, dinv, b2.reshape(1, _H),
                       batch.reshape(_N, 1), Wc1, bc1.reshape(1, _FH),
                       Wc2, bc2.reshape(1, _C))


# speedup vs baseline: 26.7670x; 1.5217x over previous
import functools

import jax
import jax.numpy as jnp
from jax import lax
from jax.experimental import pallas as pl
from jax.experimental.pallas import tpu as pltpu
from jax.experimental.pallas import tpu_sc as plsc

_N = 10000
_E = 320000
_D = 128
_H = 128
_HD = _H // 2
_FH = 256
_C = 5
_G = 64

_NSC = 2
_NTILE = 16
_NW = _NSC * _NTILE

_CH = 125
_NP = 10240
_RPT = _NP // _NTILE
_ZR = 128

_EPW_D = _E // _NW
_NCHUNK_D = _EPW_D // _CH
_EPT = _E // _NTILE
_NCHUNK = _EPT // _CH

_R = 2000
_NBLK = _N // _R

_sc_mesh = plsc.VectorSubcoreMesh(core_axis_name="c", subcore_axis_name="s")


@functools.partial(
    pl.kernel,
    out_type=jax.ShapeDtypeStruct((_NSC, _NP, 16), jnp.float32),
    mesh=_sc_mesh,
    compiler_params=pltpu.CompilerParams(use_tc_tiling_on_sc=False),
    scratch_types=[
        pltpu.VMEM((_NCHUNK_D, _CH), jnp.int32),
        pltpu.VMEM((_CH, 16), jnp.float32),
        pltpu.VMEM((_ZR, 16), jnp.float32),
        pltpu.VMEM_SHARED((_NP, 16), jnp.float32),
    ],
)
def _deg(dst_hbm, out_hbm, dstb, onesb, zb, acc):
    c = lax.axis_index("c")
    s = lax.axis_index("s")
    wid = c * _NTILE + s

    pltpu.sync_copy(dst_hbm.at[wid], dstb)

    def _fill(i, carry):
        onesb[i, :] = jnp.full((16,), 1.0, jnp.float32)
        zb[i, :] = jnp.zeros((16,), jnp.float32)
        return carry

    lax.fori_loop(0, _CH, _fill, 0)

    def _zero(k, carry):
        pltpu.sync_copy(zb, acc.at[pl.ds(s * _RPT + k * _ZR, _ZR)])
        return carry

    lax.fori_loop(0, _RPT // _ZR, _zero, 0)
    plsc.subcore_barrier()

    def _step(g, carry):
        pltpu.sync_copy(onesb, acc.at[dstb.at[g]], add=True)
        return carry

    lax.fori_loop(0, _NCHUNK_D, _step, 0)
    plsc.subcore_barrier()
    pltpu.sync_copy(acc.at[pl.ds(s * _RPT, _RPT)],
                    out_hbm.at[c, pl.ds(s * _RPT, _RPT)])


@functools.partial(
    pl.kernel,
    out_type=jax.ShapeDtypeStruct((_NSC, _NP, _HD), jnp.float32),
    mesh=_sc_mesh,
    compiler_params=pltpu.CompilerParams(use_tc_tiling_on_sc=False),
    scratch_types=[
        pltpu.VMEM((_NCHUNK, _CH), jnp.int32),
        pltpu.VMEM((_NCHUNK, _CH), jnp.int32),
        pltpu.VMEM((_CH, _HD), jnp.float32),
        pltpu.VMEM((_CH, _HD), jnp.float32),
        pltpu.VMEM((_ZR, _HD), jnp.float32),
        pltpu.VMEM_SHARED((_NP, _HD), jnp.float32),
        pltpu.SemaphoreType.DMA,
        pltpu.SemaphoreType.DMA,
        pltpu.SemaphoreType.DMA,
        pltpu.SemaphoreType.DMA,
    ],
)
def _mp(hp_hbm, src_hbm, dst_hbm, out_hbm, srcb, dstb, rows0, rows1, zb,
        acc, semg0, semg1, sems0, sems1):
    c = lax.axis_index("c")
    s = lax.axis_index("s")

    pltpu.sync_copy(src_hbm.at[s], srcb)
    pltpu.sync_copy(dst_hbm.at[s], dstb)

    def _fill(i, carry):
        for j in range(_HD // 16):
            zb[i, pl.ds(j * 16, 16)] = jnp.zeros((16,), jnp.float32)
        return carry

    lax.fori_loop(0, _ZR, _fill, 0)

    def _zero(k, carry):
        pltpu.sync_copy(zb, acc.at[pl.ds(s * _RPT + k * _ZR, _ZR)])
        return carry

    lax.fori_loop(0, _RPT // _ZR, _zero, 0)
    plsc.subcore_barrier()

    _K2 = _NCHUNK // 2
    pltpu.async_copy(hp_hbm.at[c].at[srcb.at[0]], rows0, semg0)

    def _pair(k, carry):
        g0 = 2 * k
        g1 = 2 * k + 1

        @pl.when(k > 0)
        def _():
            pltpu.make_async_copy(rows1, acc.at[dstb.at[g1]], sems1).wait()

        pltpu.async_copy(hp_hbm.at[c].at[srcb.at[g1]], rows1, semg1)
        pltpu.make_async_copy(hp_hbm.at[c].at[srcb.at[g0]], rows0,
                              semg0).wait()
        pltpu.async_copy(rows0, acc.at[dstb.at[g0]], sems0, add=True)

        @pl.when(k < _K2 - 1)
        def _():
            pltpu.make_async_copy(rows0, acc.at[dstb.at[g0]], sems0).wait()
            pltpu.async_copy(hp_hbm.at[c].at[srcb.at[g0 + 2]], rows0, semg0)

        pltpu.make_async_copy(hp_hbm.at[c].at[srcb.at[g1]], rows1,
                              semg1).wait()
        pltpu.async_copy(rows1, acc.at[dstb.at[g1]], sems1, add=True)
        return carry

    lax.fori_loop(0, _K2, _pair, 0)
    pltpu.make_async_copy(rows0, acc.at[dstb.at[0]], sems0).wait()
    pltpu.make_async_copy(rows1, acc.at[dstb.at[0]], sems1).wait()
    plsc.subcore_barrier()
    pltpu.sync_copy(acc.at[pl.ds(s * _RPT, _RPT)],
                    out_hbm.at[c, pl.ds(s * _RPT, _RPT)])


def _prep_body(x_ref, w_ref, degp_ref, hp_ref, dinv_ref):
    deg = degp_ref[0, :, 0:1] + degp_ref[1, :, 0:1] + 1.0
    dinv = lax.rsqrt(deg)
    h = jnp.dot(x_ref[...], w_ref[...], preferred_element_type=jnp.float32)
    hp = h * dinv
    hp_ref[0] = hp[:, 0:_HD]
    hp_ref[1] = hp[:, _HD:_H]
    dinv_ref[...] = dinv


def _prep_call(x, W1, degp):
    return pl.pallas_call(
        _prep_body,
        grid=(_NBLK,),
        in_specs=[
            pl.BlockSpec((_R, _D), lambda i: (i, 0)),
            pl.BlockSpec((_D, _H), lambda i: (0, 0)),
            pl.BlockSpec((_NSC, _R, 16), lambda i: (0, i, 0)),
        ],
        out_specs=[
            pl.BlockSpec((_NSC, _R, _HD), lambda i: (0, i, 0)),
            pl.BlockSpec((_R, 1), lambda i: (i, 0)),
        ],
        out_shape=[
            jax.ShapeDtypeStruct((_NSC, _N, _HD), jnp.float32),
            jax.ShapeDtypeStruct((_N, 1), jnp.float32),
        ],
    )(x, W1, degp)


def _mid_body(acc_ref, hp_ref, dinv_ref, b_ref, w_ref, out_ref):
    t = jnp.concatenate([acc_ref[0] + hp_ref[0], acc_ref[1] + hp_ref[1]],
                        axis=1)
    o = jnp.maximum(t * dinv_ref[...] + b_ref[...], 0.0)
    h2 = jnp.dot(o, w_ref[...], preferred_element_type=jnp.float32)
    hp2 = h2 * dinv_ref[...]
    out_ref[0] = hp2[:, 0:_HD]
    out_ref[1] = hp2[:, _HD:_H]


def _mid_call(acc, hp1, dinv, b1, W2):
    return pl.pallas_call(
        _mid_body,
        grid=(_NBLK,),
        in_specs=[
            pl.BlockSpec((_NSC, _R, _HD), lambda i: (0, i, 0)),
            pl.BlockSpec((_NSC, _R, _HD), lambda i: (0, i, 0)),
            pl.BlockSpec((_R, 1), lambda i: (i, 0)),
            pl.BlockSpec((1, _H), lambda i: (0, 0)),
            pl.BlockSpec((_H, _H), lambda i: (0, 0)),
        ],
        out_specs=pl.BlockSpec((_NSC, _R, _HD), lambda i: (0, i, 0)),
        out_shape=jax.ShapeDtypeStruct((_NSC, _N, _HD), jnp.float32),
    )(acc, hp1, dinv, b1, W2)


def _final_body(acc_ref, hp_ref, dinv_ref, b_ref, batch_ref, wc1_ref,
                bc1_ref, wc2_ref, bc2_ref, out_ref, pooled, counts):
    i = pl.program_id(0)

    @pl.when(i == 0)
    def _():
        pooled[...] = jnp.zeros_like(pooled)
        counts[...] = jnp.zeros_like(counts)

    t = jnp.concatenate([acc_ref[0] + hp_ref[0], acc_ref[1] + hp_ref[1]],
                        axis=1)
    o = jnp.maximum(t * dinv_ref[...] + b_ref[...], 0.0)
    gids = lax.broadcasted_iota(jnp.int32, (_R, _G), 1)
    mask = (batch_ref[...] == gids).astype(jnp.float32)
    dn = (((0,), (0,)), ((), ()))
    pooled[...] += lax.dot_general(mask, o, dn,
                                   preferred_element_type=jnp.float32)
    counts[...] += lax.dot_general(mask, jnp.ones((_R, 1), jnp.float32), dn,
                                   preferred_element_type=jnp.float32)

    @pl.when(i == pl.num_programs(0) - 1)
    def _():
        pm = pooled[...] / jnp.maximum(counts[...], 1.0)
        z = jnp.maximum(
            jnp.dot(pm, wc1_ref[...], preferred_element_type=jnp.float32)
            + bc1_ref[...], 0.0)
        out_ref[...] = (jnp.dot(z, wc2_ref[...],
                                preferred_element_type=jnp.float32)
                        + bc2_ref[...])


def _final_call(acc, hp2, dinv, b2, batch2, Wc1, bc1, Wc2, bc2):
    return pl.pallas_call(
        _final_body,
        grid=(_NBLK,),
        in_specs=[
            pl.BlockSpec((_NSC, _R, _HD), lambda i: (0, i, 0)),
            pl.BlockSpec((_NSC, _R, _HD), lambda i: (0, i, 0)),
            pl.BlockSpec((_R, 1), lambda i: (i, 0)),
            pl.BlockSpec((1, _H), lambda i: (0, 0)),
            pl.BlockSpec((_R, 1), lambda i: (i, 0)),
            pl.BlockSpec((_H, _FH), lambda i: (0, 0)),
            pl.BlockSpec((1, _FH), lambda i: (0, 0)),
            pl.BlockSpec((_FH, _C), lambda i: (0, 0)),
            pl.BlockSpec((1, _C), lambda i: (0, 0)),
        ],
        out_specs=pl.BlockSpec((_G, _C), lambda i: (0, 0)),
        out_shape=jax.ShapeDtypeStruct((_G, _C), jnp.float32),
        scratch_shapes=[
            pltpu.VMEM((_G, _H), jnp.float32),
            pltpu.VMEM((_G, 1), jnp.float32),
        ],
    )(acc, hp2, dinv, b2, batch2, Wc1, bc1, Wc2, bc2)


def kernel(x, edge_index, batch, W1, b1, W2, b2, Wc1, bc1, Wc2, bc2):
    src_t = edge_index[0].reshape(_NTILE, _NCHUNK, _CH)
    dst_t = edge_index[1].reshape(_NTILE, _NCHUNK, _CH)
    dst_w = edge_index[1].reshape(_NW, _NCHUNK_D, _CH)

    degp = _deg(dst_w)
    hp1, dinv = _prep_call(x, W1, degp)
    acc1 = _mp(hp1, src_t, dst_t)
    hp2 = _mid_call(acc1, hp1, dinv, b1.reshape(1, _H), W2)
    acc2 = _mp(hp2, src_t, dst_t)
    return _final_call(acc2, hp2, dinv, b2.reshape(1, _H),
                       batch.reshape(_N, 1), Wc1, bc1.reshape(1, _FH),
                       Wc2, bc2.reshape(1, _C))
